# baseline (device time: 24681 ns/iter reference)
import jax
import jax.numpy as jnp
from jax import lax
from jax.experimental import pallas as pl
from jax.experimental.pallas import tpu as pltpu

N_DEV = 4
N_TOK = 512
D_IN = 256
D_OUT = 512
E_LOCAL = 4
N_EXP = 16
BLK = N_TOK // N_DEV


def kernel(x, router_W, route_idx, expert_W, shared_W):
    def body(x_ref, rw_ref, idx_ref, ew_ref, sw_ref, out_ref,
             p_ref, comm_ref, send_sems, recv_sems):
        my_pos = lax.axis_index("i")
        left = lax.rem(my_pos + N_DEV - 1, N_DEV)
        right = lax.rem(my_pos + 1, N_DEV)

        barrier_sem = pltpu.get_barrier_semaphore()
        for nbr in (left, right):
            pl.semaphore_signal(barrier_sem, inc=1, device_id=(nbr,),
                                device_id_type=pl.DeviceIdType.MESH)
        pl.semaphore_wait(barrier_sem, 2)

        xv = x_ref[:, :]
        scores = jnp.dot(xv, rw_ref[:, :], preferred_element_type=jnp.float32)
        s_max = jnp.max(scores, axis=-1, keepdims=True)
        ex = jnp.exp(scores - s_max)
        probs = ex / jnp.sum(ex, axis=-1, keepdims=True)
        ridx = idx_ref[:, :]
        col = lax.broadcasted_iota(jnp.int32, (N_TOK, N_EXP), 1)
        p_sel = jnp.sum(jnp.where(col == ridx, probs, 0.0),
                        axis=1, keepdims=True)

        acc = jnp.zeros((N_TOK, D_OUT), jnp.float32)
        for le in range(E_LOCAL):
            ge = my_pos * E_LOCAL + le
            coef = jnp.where(ridx == ge, p_sel, 0.0)
            acc = acc + jnp.dot(xv * coef, ew_ref[le],
                                preferred_element_type=jnp.float32)
        p_ref[:, :] = acc

        for h in range(N_DEV - 1):
            d = lax.rem(my_pos + 2 * N_DEV - 1 - h, N_DEV)
            blk = p_ref[pl.ds(d * BLK, BLK), :]
            if h == 0:
                comm_ref[0, :, :] = blk
            else:
                comm_ref[h, :, :] = comm_ref[h, :, :] + blk
            rdma = pltpu.make_async_remote_copy(
                src_ref=comm_ref.at[h],
                dst_ref=comm_ref.at[h + 1],
                send_sem=send_sems.at[h],
                recv_sem=recv_sems.at[h],
                device_id=(right,),
                device_id_type=pl.DeviceIdType.MESH,
            )
            rdma.start()
            rdma.wait()

        shared_blk = jnp.dot(x_ref[pl.ds(my_pos * BLK, BLK), :], sw_ref[:, :],
                             preferred_element_type=jnp.float32)
        out_ref[:, :] = (shared_blk + comm_ref[N_DEV - 1, :, :]
                         + p_ref[pl.ds(my_pos * BLK, BLK), :])

    return pl.pallas_call(
        body,
        out_shape=jax.ShapeDtypeStruct((BLK, D_OUT), jnp.float32),
        in_specs=[pl.BlockSpec(memory_space=pltpu.VMEM)] * 5,
        out_specs=pl.BlockSpec(memory_space=pltpu.VMEM),
        scratch_shapes=[
            pltpu.VMEM((N_TOK, D_OUT), jnp.float32),
            pltpu.VMEM((N_DEV, BLK, D_OUT), jnp.float32),
            pltpu.SemaphoreType.DMA((N_DEV - 1,)),
            pltpu.SemaphoreType.DMA((N_DEV - 1,)),
        ],
        compiler_params=pltpu.CompilerParams(collective_id=0),
    )(x, router_W, route_idx, expert_W, shared_W)


# device time: 18273 ns/iter; 1.3507x vs baseline; 1.3507x over previous
import jax
import jax.numpy as jnp
from jax import lax
from jax.experimental import pallas as pl
from jax.experimental.pallas import tpu as pltpu

N_DEV = 4
N_TOK = 512
D_IN = 256
D_OUT = 512
E_LOCAL = 4
N_EXP = 16
BLK = N_TOK // N_DEV


def kernel(x, router_W, route_idx, expert_W, shared_W):
    def body(x_ref, rw_ref, idx_ref, ew_ref, sw_ref, out_ref,
             coef_ref, send_bufs, recv_bufs, send_sems, recv_sems):
        my_pos = lax.axis_index("i")

        barrier_sem = pltpu.get_barrier_semaphore()
        for m in range(1, N_DEV):
            peer = lax.rem(my_pos + m, N_DEV)
            pl.semaphore_signal(barrier_sem, inc=1, device_id=(peer,),
                                device_id_type=pl.DeviceIdType.MESH)
        pl.semaphore_wait(barrier_sem, N_DEV - 1)

        xv = x_ref[:, :]
        scores = jnp.dot(xv, rw_ref[:, :], preferred_element_type=jnp.float32)
        s_max = jnp.max(scores, axis=-1, keepdims=True)
        ex = jnp.exp(scores - s_max)
        probs = ex / jnp.sum(ex, axis=-1, keepdims=True)
        ridx = idx_ref[:, :]
        col = lax.broadcasted_iota(jnp.int32, (N_TOK, N_EXP), 1)
        p_sel = jnp.sum(jnp.where(col == ridx, probs, 0.0),
                        axis=1, keepdims=True)
        lexp = lax.broadcasted_iota(jnp.int32, (N_TOK, E_LOCAL), 1)
        coef_ref[:, :] = jnp.where(ridx == my_pos * E_LOCAL + lexp,
                                   p_sel, 0.0)

        def partial_block(d):
            rows = pl.ds(d * BLK, BLK)
            xb = x_ref[rows, :]
            cb = coef_ref[rows, :]
            acc = jnp.zeros((BLK, D_OUT), jnp.float32)
            for le in range(E_LOCAL):
                acc = acc + jnp.dot(xb * cb[:, le:le + 1], ew_ref[le],
                                    preferred_element_type=jnp.float32)
            return acc

        rdmas = []
        for m in range(1, N_DEV):
            d = lax.rem(my_pos + m, N_DEV)
            slot = N_DEV - 1 - m
            send_bufs[m - 1, :, :] = partial_block(d)
            rdma = pltpu.make_async_remote_copy(
                src_ref=send_bufs.at[m - 1],
                dst_ref=recv_bufs.at[slot],
                send_sem=send_sems.at[m - 1],
                recv_sem=recv_sems.at[slot],
                device_id=(d,),
                device_id_type=pl.DeviceIdType.MESH,
            )
            rdma.start()
            rdmas.append(rdma)

        shared_blk = jnp.dot(x_ref[pl.ds(my_pos * BLK, BLK), :], sw_ref[:, :],
                             preferred_element_type=jnp.float32)
        own = shared_blk + partial_block(my_pos)

        for j in range(N_DEV - 1):
            recv = pltpu.make_async_remote_copy(
                src_ref=send_bufs.at[0],
                dst_ref=recv_bufs.at[j],
                send_sem=send_sems.at[0],
                recv_sem=recv_sems.at[j],
                device_id=(my_pos,),
                device_id_type=pl.DeviceIdType.MESH,
            )
            recv.wait_recv()
        out_ref[:, :] = (own + recv_bufs[0, :, :] + recv_bufs[1, :, :]
                         + recv_bufs[2, :, :])

        for rdma in rdmas:
            rdma.wait_send()

    return pl.pallas_call(
        body,
        out_shape=jax.ShapeDtypeStruct((BLK, D_OUT), jnp.float32),
        in_specs=[pl.BlockSpec(memory_space=pltpu.VMEM)] * 5,
        out_specs=pl.BlockSpec(memory_space=pltpu.VMEM),
        scratch_shapes=[
            pltpu.VMEM((N_TOK, E_LOCAL), jnp.float32),
            pltpu.VMEM((N_DEV - 1, BLK, D_OUT), jnp.float32),
            pltpu.VMEM((N_DEV - 1, BLK, D_OUT), jnp.float32),
            pltpu.SemaphoreType.DMA((N_DEV - 1,)),
            pltpu.SemaphoreType.DMA((N_DEV - 1,)),
        ],
        compiler_params=pltpu.CompilerParams(collective_id=0),
    )(x, router_W, route_idx, expert_W, shared_W)


# device time: 15050 ns/iter; 1.6399x vs baseline; 1.2142x over previous
import jax
import jax.numpy as jnp
from jax import lax
from jax.experimental import pallas as pl
from jax.experimental.pallas import tpu as pltpu

N_DEV = 4
N_TOK = 512
D_IN = 256
D_OUT = 512
E_LOCAL = 4
N_EXP = 16
BLK = N_TOK // N_DEV


def kernel(x, router_W, route_idx, expert_W, shared_W):
    def body(x_ref, rw_ref, idx_ref, ew_ref, sw_ref, out_ref,
             coef_ref, xbf_ref, send_bufs, recv_bufs, send_sems, recv_sems):
        my_pos = lax.axis_index("i")

        barrier_sem = pltpu.get_barrier_semaphore()
        for m in range(1, N_DEV):
            peer = lax.rem(my_pos + m, N_DEV)
            pl.semaphore_signal(barrier_sem, inc=1, device_id=(peer,),
                                device_id_type=pl.DeviceIdType.MESH)
        pl.semaphore_wait(barrier_sem, N_DEV - 1)

        xv = x_ref[:, :]
        scores = jnp.dot(xv, rw_ref[:, :], preferred_element_type=jnp.float32)
        s_max = jnp.max(scores, axis=-1, keepdims=True)
        ex = jnp.exp(scores - s_max)
        probs = ex / jnp.sum(ex, axis=-1, keepdims=True)
        ridx = idx_ref[:, :]
        col = lax.broadcasted_iota(jnp.int32, (N_TOK, N_EXP), 1)
        p_sel = jnp.sum(jnp.where(col == ridx, probs, 0.0),
                        axis=1, keepdims=True)
        lexp = lax.broadcasted_iota(jnp.int32, (N_TOK, E_LOCAL), 1)
        coef_ref[:, :] = jnp.where(ridx == my_pos * E_LOCAL + lexp,
                                   p_sel, 0.0)

        xbf_ref[:, :] = xv.astype(jnp.bfloat16)
        ewbf = ew_ref[:, :, :].astype(jnp.bfloat16)

        def partial_block(d):
            rows = pl.ds(d * BLK, BLK)
            xb = xbf_ref[rows, :]
            cb = coef_ref[rows, :]
            acc = jnp.zeros((BLK, D_OUT), jnp.float32)
            for le in range(E_LOCAL):
                g = jnp.dot(xb, ewbf[le],
                            preferred_element_type=jnp.float32)
                acc = acc + cb[:, le:le + 1] * g
            return acc

        rdmas = []
        for m in range(1, N_DEV):
            d = lax.rem(my_pos + m, N_DEV)
            slot = N_DEV - 1 - m
            send_bufs[m - 1, :, :] = partial_block(d).astype(jnp.bfloat16)
            rdma = pltpu.make_async_remote_copy(
                src_ref=send_bufs.at[m - 1],
                dst_ref=recv_bufs.at[slot],
                send_sem=send_sems.at[m - 1],
                recv_sem=recv_sems.at[slot],
                device_id=(d,),
                device_id_type=pl.DeviceIdType.MESH,
            )
            rdma.start()
            rdmas.append(rdma)

        shared_blk = jnp.dot(xbf_ref[pl.ds(my_pos * BLK, BLK), :],
                             sw_ref[:, :].astype(jnp.bfloat16),
                             preferred_element_type=jnp.float32)
        own = shared_blk + partial_block(my_pos)

        for j in range(N_DEV - 1):
            recv = pltpu.make_async_remote_copy(
                src_ref=send_bufs.at[0],
                dst_ref=recv_bufs.at[j],
                send_sem=send_sems.at[0],
                recv_sem=recv_sems.at[j],
                device_id=(my_pos,),
                device_id_type=pl.DeviceIdType.MESH,
            )
            recv.wait_recv()
        out_ref[:, :] = (own
                         + recv_bufs[0, :, :].astype(jnp.float32)
                         + recv_bufs[1, :, :].astype(jnp.float32)
                         + recv_bufs[2, :, :].astype(jnp.float32))

        for rdma in rdmas:
            rdma.wait_send()

    return pl.pallas_call(
        body,
        out_shape=jax.ShapeDtypeStruct((BLK, D_OUT), jnp.float32),
        in_specs=[pl.BlockSpec(memory_space=pltpu.VMEM)] * 5,
        out_specs=pl.BlockSpec(memory_space=pltpu.VMEM),
        scratch_shapes=[
            pltpu.VMEM((N_TOK, E_LOCAL), jnp.float32),
            pltpu.VMEM((N_TOK, D_IN), jnp.bfloat16),
            pltpu.VMEM((N_DEV - 1, BLK, D_OUT), jnp.bfloat16),
            pltpu.VMEM((N_DEV - 1, BLK, D_OUT), jnp.bfloat16),
            pltpu.SemaphoreType.DMA((N_DEV - 1,)),
            pltpu.SemaphoreType.DMA((N_DEV - 1,)),
        ],
        compiler_params=pltpu.CompilerParams(collective_id=0),
    )(x, router_W, route_idx, expert_W, shared_W)


# device time: 14885 ns/iter; 1.6581x vs baseline; 1.0111x over previous
import jax
import jax.numpy as jnp
from jax import lax
from jax.experimental import pallas as pl
from jax.experimental.pallas import tpu as pltpu

N_DEV = 4
N_TOK = 512
D_IN = 256
D_OUT = 512
E_LOCAL = 4
N_EXP = 16
BLK = N_TOK // N_DEV


def kernel(x, router_W, route_idx, expert_W, shared_W):
    def body(x_ref, rw_ref, idx_ref, ew_ref, sw_ref, out_ref,
             coef_ref, xbf_ref, send_bufs, recv_bufs, send_sems, recv_sems):
        my_pos = lax.axis_index("i")

        barrier_sem = pltpu.get_barrier_semaphore()
        for m in range(1, N_DEV):
            peer = lax.rem(my_pos + m, N_DEV)
            pl.semaphore_signal(barrier_sem, inc=1, device_id=(peer,),
                                device_id_type=pl.DeviceIdType.MESH)

        xv = x_ref[:, :]
        scores = jnp.dot(xv, rw_ref[:, :], preferred_element_type=jnp.float32)
        s_max = jnp.max(scores, axis=-1, keepdims=True)
        ex = jnp.exp(scores - s_max)
        probs = ex / jnp.sum(ex, axis=-1, keepdims=True)
        ridx = idx_ref[:, :]
        col = lax.broadcasted_iota(jnp.int32, (N_TOK, N_EXP), 1)
        p_sel = jnp.sum(jnp.where(col == ridx, probs, 0.0),
                        axis=1, keepdims=True)
        lexp = lax.broadcasted_iota(jnp.int32, (N_TOK, E_LOCAL), 1)
        coef_ref[:, :] = jnp.where(ridx == my_pos * E_LOCAL + lexp,
                                   p_sel, 0.0).astype(jnp.bfloat16)

        xbf_ref[:, :] = xv.astype(jnp.bfloat16)
        ewbf = ew_ref[:, :].astype(jnp.bfloat16)

        def partial_block(d):
            rows = pl.ds(d * BLK, BLK)
            xb = xbf_ref[rows, :]
            cb = coef_ref[rows, :]
            xstack = jnp.concatenate(
                [xb * cb[:, le:le + 1] for le in range(E_LOCAL)], axis=1)
            return jnp.dot(xstack, ewbf, preferred_element_type=jnp.float32)

        pl.semaphore_wait(barrier_sem, N_DEV - 1)

        rdmas = []
        for m in range(1, N_DEV):
            d = lax.rem(my_pos + m, N_DEV)
            slot = N_DEV - 1 - m
            send_bufs[m - 1, :, :] = partial_block(d).astype(jnp.bfloat16)
            rdma = pltpu.make_async_remote_copy(
                src_ref=send_bufs.at[m - 1],
                dst_ref=recv_bufs.at[slot],
                send_sem=send_sems.at[m - 1],
                recv_sem=recv_sems.at[slot],
                device_id=(d,),
                device_id_type=pl.DeviceIdType.MESH,
            )
            rdma.start()
            rdmas.append(rdma)

        shared_blk = jnp.dot(xbf_ref[pl.ds(my_pos * BLK, BLK), :],
                             sw_ref[:, :].astype(jnp.bfloat16),
                             preferred_element_type=jnp.float32)
        own = shared_blk + partial_block(my_pos)

        for j in range(N_DEV - 1):
            recv = pltpu.make_async_remote_copy(
                src_ref=send_bufs.at[0],
                dst_ref=recv_bufs.at[j],
                send_sem=send_sems.at[0],
                recv_sem=recv_sems.at[j],
                device_id=(my_pos,),
                device_id_type=pl.DeviceIdType.MESH,
            )
            recv.wait_recv()
        out_ref[:, :] = (own
                         + recv_bufs[0, :, :].astype(jnp.float32)
                         + recv_bufs[1, :, :].astype(jnp.float32)
                         + recv_bufs[2, :, :].astype(jnp.float32))

        for rdma in rdmas:
            rdma.wait_send()

    return pl.pallas_call(
        body,
        out_shape=jax.ShapeDtypeStruct((BLK, D_OUT), jnp.float32),
        in_specs=[pl.BlockSpec(memory_space=pltpu.VMEM)] * 5,
        out_specs=pl.BlockSpec(memory_space=pltpu.VMEM),
        scratch_shapes=[
            pltpu.VMEM((N_TOK, E_LOCAL), jnp.bfloat16),
            pltpu.VMEM((N_TOK, D_IN), jnp.bfloat16),
            pltpu.VMEM((N_DEV - 1, BLK, D_OUT), jnp.bfloat16),
            pltpu.VMEM((N_DEV - 1, BLK, D_OUT), jnp.bfloat16),
            pltpu.SemaphoreType.DMA((N_DEV - 1,)),
            pltpu.SemaphoreType.DMA((N_DEV - 1,)),
        ],
        compiler_params=pltpu.CompilerParams(collective_id=0),
    )(x, router_W, route_idx, expert_W.reshape(E_LOCAL * D_IN, D_OUT),
      shared_W)
